# baseline (speedup 1.0000x reference)
import functools

import jax
import jax.numpy as jnp
from jax import lax
from jax.experimental import pallas as pl
from jax.experimental.pallas import tpu as pltpu
from jax.experimental.pallas import tpu_sc as plsc

R = 4096
C = 8192
NC = 2
NS = 16
L = 16
NW = NC * NS
ROWS_PER_W = R // NW
TR = 8
TCW = 128
TW = TR * TCW
NTR = ROWS_PER_W // TR
NCHAIN = TR
CHUNK = TCW
PADTW = 1029
NCHUNK = C // CHUNK
NPAIR = NCHUNK // 2


def _cumsum_body(x_hbm, o_hbm, ia, ib, oa, ob, sia, sib, soa, sob):
    cid = lax.axis_index("c")
    sid = lax.axis_index("s")
    wid = sid * NC + cid
    tr0 = wid * NTR

    lane = lax.iota(jnp.int32, L)

    def in_copy(buf, sem, ch):
        return pltpu.make_async_copy(
            x_hbm.at[pl.ds(tr0, NTR), ch],
            buf.at[:, pl.ds(0, TW)],
            sem,
        )

    def out_copy(buf, sem, ch):
        return pltpu.make_async_copy(
            buf.at[:, pl.ds(0, TW)],
            o_hbm.at[pl.ds(tr0, NTR), ch],
            sem,
        )

    def compute(src, dst, accs):
        def body(c, st):
            accs, col = st
            idx = [col + j * TCW for j in range(NCHAIN)]
            vals = [plsc.load_gather(src, [lane, idx[j]]) for j in range(NCHAIN)]
            new = [accs[j] + vals[j] for j in range(NCHAIN)]
            for j in range(NCHAIN):
                plsc.store_scatter(dst, [lane, idx[j]], new[j])
            return (tuple(new), col + 1)

        st = (accs, jnp.zeros((L,), jnp.int32))
        accs, _ = plsc.parallel_loop(0, CHUNK, carry=st, unroll=2)(body)
        return accs

    in_copy(ia, sia, 0).start()

    def pair(i, accs):
        ka = 2 * i
        in_copy(ia, sia, ka).wait()
        in_copy(ib, sib, ka + 1).start()

        @pl.when(i > 0)
        def _():
            out_copy(oa, soa, ka - 2).wait()

        accs = compute(ia, oa, accs)
        out_copy(oa, soa, ka).start()

        in_copy(ib, sib, ka + 1).wait()

        @pl.when(i < NPAIR - 1)
        def _():
            in_copy(ia, sia, ka + 2).start()

        @pl.when(i > 0)
        def _():
            out_copy(ob, sob, ka - 1).wait()

        accs = compute(ib, ob, accs)
        out_copy(ob, sob, ka + 1).start()
        return accs

    accs = tuple(jnp.zeros((L,), jnp.float32) for _ in range(NCHAIN))
    lax.fori_loop(0, NPAIR, pair, accs)
    out_copy(oa, soa, NCHUNK - 2).wait()
    out_copy(ob, sob, NCHUNK - 1).wait()


def _make_kernel():
    mesh = plsc.VectorSubcoreMesh(core_axis_name="c", subcore_axis_name="s")
    return functools.partial(
        pl.kernel,
        mesh=mesh,
        out_type=jax.ShapeDtypeStruct((R // TR, C // TCW, TW), jnp.float32),
        scratch_types=[
            pltpu.VMEM((NTR, PADTW), jnp.float32),
            pltpu.VMEM((NTR, PADTW), jnp.float32),
            pltpu.VMEM((NTR, PADTW), jnp.float32),
            pltpu.VMEM((NTR, PADTW), jnp.float32),
            pltpu.SemaphoreType.DMA,
            pltpu.SemaphoreType.DMA,
            pltpu.SemaphoreType.DMA,
            pltpu.SemaphoreType.DMA,
        ],
        compiler_params=pltpu.CompilerParams(
            use_tc_tiling_on_sc=False, needs_layout_passes=False
        ),
    )(_cumsum_body)


_sc_cumsum = _make_kernel()


def kernel(x):
    x3 = x.astype(jnp.float32).reshape(R // TR, TR, C // TCW, TCW)
    x3 = x3.transpose(0, 2, 1, 3).reshape(R // TR, C // TCW, TW)
    o3 = _sc_cumsum(x3)
    o4 = o3.reshape(R // TR, C // TCW, TR, TCW).transpose(0, 2, 1, 3)
    return o4.reshape(R, C)
